# Initial kernel scaffold; baseline (speedup 1.0000x reference)
import jax
import jax.numpy as jnp
from jax.experimental import pallas as pl


def kernel(q_sub, hidden, edges, n_node, old_nodes_new_idx, entity_pretrain_emb, h_sub, rela_embed, W1, Ws_attn, Wr_attn, w_alpha_w, w_alpha_b, W_h):
    raise NotImplementedError("write your pallas kernel here")


# speedup vs baseline: 3.6218x; 3.6218x over previous
import functools

import jax
import jax.numpy as jnp
from jax import lax
from jax.experimental import pallas as pl
from jax.experimental.pallas import tpu as pltpu
from jax.experimental.pallas import tpu_sc as plsc

N_NODES = 10000
N_EDGES = 320000
DIM = 128
ADIM = 64

NC = 2
NS = 16
NW = NC * NS
EPW = N_EDGES // NW
BC = 80
NCH = EPW // BC
NPAD = 10240
RPT = NPAD // NS

_mesh = plsc.VectorSubcoreMesh(core_axis_name="c", subcore_axis_name="s")


def _wid():
    return lax.axis_index("s") * NC + lax.axis_index("c")


@functools.partial(
    pl.kernel,
    mesh=_mesh,
    out_type=jax.ShapeDtypeStruct((N_EDGES, DIM), jnp.float32),
    scratch_types=[
        pltpu.VMEM((NCH, BC), jnp.int32),
        pltpu.VMEM((NCH, BC), jnp.int32),
        pltpu.VMEM((BC, DIM), jnp.float32),
        pltpu.VMEM((BC, DIM), jnp.float32),
        pltpu.SemaphoreType.DMA,
    ],
)
def _gather_sum(hid_hbm, rel_hbm, subi_hbm, reli_hbm, g_hbm,
                subbuf, relbuf, hbuf, rbuf, sem):
    w = _wid()
    pltpu.sync_copy(subi_hbm.at[w], subbuf)
    pltpu.sync_copy(reli_hbm.at[w], relbuf)

    def chunk(c, _):
        d1 = pltpu.async_copy(hid_hbm.at[subbuf.at[c]], hbuf, sem)
        d2 = pltpu.async_copy(rel_hbm.at[relbuf.at[c]], rbuf, sem)
        d1.wait()
        d2.wait()

        def row(i, _):
            for j in range(DIM // 16):
                sl = pl.ds(j * 16, 16)
                hbuf[i, sl] = hbuf[i, sl] + rbuf[i, sl]
            return _

        lax.fori_loop(0, BC, row, 0)
        pltpu.sync_copy(hbuf, g_hbm.at[pl.ds(w * EPW + c * BC, BC)])
        return _

    lax.fori_loop(0, NCH, chunk, 0)


@functools.partial(
    pl.kernel,
    mesh=_mesh,
    out_type=jax.ShapeDtypeStruct((NC, NPAD, DIM), jnp.float32),
    scratch_types=[
        pltpu.VMEM((NCH, BC), jnp.int32),
        pltpu.VMEM((BC, DIM), jnp.float32),
        pltpu.VMEM((BC, DIM), jnp.float32),
        pltpu.VMEM_SHARED((NPAD, DIM), jnp.float32),
    ],
)
def _scatter_add(xw_hbm, obji_hbm, z_hbm, objbuf, xwbuf, zerobuf, zsh):
    cid = lax.axis_index("c")
    sid = lax.axis_index("s")
    w = _wid()

    def zrow(i, _):
        for j in range(DIM // 16):
            zerobuf[i, pl.ds(j * 16, 16)] = jnp.zeros((16,), jnp.float32)
        return _

    lax.fori_loop(0, BC, zrow, 0)

    def zcopy(r, _):
        pltpu.sync_copy(zerobuf, zsh.at[pl.ds(sid * RPT + r * BC, BC)])
        return _

    lax.fori_loop(0, RPT // BC, zcopy, 0)
    pltpu.sync_copy(obji_hbm.at[w], objbuf)
    plsc.subcore_barrier()

    def chunk(c, _):
        pltpu.sync_copy(xw_hbm.at[pl.ds(w * EPW + c * BC, BC)], xwbuf)
        pltpu.sync_copy(xwbuf, zsh.at[objbuf.at[c]], add=True)
        return _

    lax.fori_loop(0, NCH, chunk, 0)
    plsc.subcore_barrier()
    pltpu.sync_copy(zsh.at[pl.ds(sid * RPT, RPT)],
                    z_hbm.at[cid, pl.ds(sid * RPT, RPT)])


def _weights_body(w1_ref, wsa_ref, wh_ref, a_ref, c_ref):
    w1 = w1_ref[...]
    a_ref[...] = jnp.dot(w1, wsa_ref[...], preferred_element_type=jnp.float32)
    c_ref[...] = jnp.dot(w1, wh_ref[...], preferred_element_type=jnp.float32)


def _edge_body(hs_ref, g_ref, a_ref, w_ref, b_ref, alpha_ref, xw_ref):
    x = hs_ref[...] + g_ref[...]
    u = jnp.dot(x, a_ref[...], preferred_element_type=jnp.float32)
    t = jnp.maximum(u, 0.0)
    logit = jnp.dot(t, w_ref[...], preferred_element_type=jnp.float32) + b_ref[0, 0]
    alpha = 1.0 / (1.0 + jnp.exp(-logit))
    alpha_ref[...] = alpha
    xw_ref[...] = x * alpha


def _final_body(z_ref, c_ref, out_ref):
    z = z_ref[0] + z_ref[1]
    out_ref[...] = jnp.dot(z, c_ref[...], preferred_element_type=jnp.float32)


BE = 4000


def kernel(q_sub, hidden, edges, n_node, old_nodes_new_idx, entity_pretrain_emb,
           h_sub, rela_embed, W1, Ws_attn, Wr_attn, w_alpha_w, w_alpha_b, W_h):
    sub = edges[:, 4].reshape(NW, NCH, BC)
    rel = edges[:, 2].reshape(NW, NCH, BC)
    obj = edges[:, 5].reshape(NW, NCH, BC)

    a_mat, c_mat = pl.pallas_call(
        _weights_body,
        out_shape=(
            jax.ShapeDtypeStruct((DIM, ADIM), jnp.float32),
            jax.ShapeDtypeStruct((DIM, DIM), jnp.float32),
        ),
    )(W1, Ws_attn, W_h)

    g = _gather_sum(hidden, rela_embed, sub, rel)

    alpha, xw = pl.pallas_call(
        _edge_body,
        grid=(N_EDGES // BE,),
        in_specs=[
            pl.BlockSpec((BE, DIM), lambda i: (i, 0)),
            pl.BlockSpec((BE, DIM), lambda i: (i, 0)),
            pl.BlockSpec((DIM, ADIM), lambda i: (0, 0)),
            pl.BlockSpec((ADIM, 1), lambda i: (0, 0)),
            pl.BlockSpec((1, 1), lambda i: (0, 0)),
        ],
        out_specs=[
            pl.BlockSpec((BE, 1), lambda i: (i, 0)),
            pl.BlockSpec((BE, DIM), lambda i: (i, 0)),
        ],
        out_shape=(
            jax.ShapeDtypeStruct((N_EDGES, 1), jnp.float32),
            jax.ShapeDtypeStruct((N_EDGES, DIM), jnp.float32),
        ),
    )(h_sub, g, a_mat, w_alpha_w, w_alpha_b.reshape(1, 1))

    zpart = _scatter_add(xw, obj)

    RB = 1000
    hidden_new = pl.pallas_call(
        _final_body,
        grid=(N_NODES // RB,),
        in_specs=[
            pl.BlockSpec((NC, RB, DIM), lambda i: (0, i, 0)),
            pl.BlockSpec((DIM, DIM), lambda i: (0, 0)),
        ],
        out_specs=pl.BlockSpec((RB, DIM), lambda i: (i, 0)),
        out_shape=jax.ShapeDtypeStruct((N_NODES, DIM), jnp.float32),
    )(zpart, c_mat)

    return (hidden_new, alpha)
